# baseline (speedup 1.0000x reference)
import functools

import jax
import jax.numpy as jnp
from jax.experimental import pallas as pl
from jax.experimental.pallas import tpu as pltpu

_SL = 8
_BB = 8
_BISECT = 34
_F32 = jnp.float32


def _t32(col):
    i = jax.lax.broadcasted_iota(jnp.int32, (32, 32), 0)
    j = jax.lax.broadcasted_iota(jnp.int32, (32, 32), 1)
    return jnp.sum(jnp.where(i == j, col, 0.0), axis=0, keepdims=True)


def _red2(x, op):
    return op(op(x, axis=2), axis=1, keepdims=True)


def _focal_parts(p):
    logits = jnp.log(p / (1.0 - p + 1e-10))
    sp = jax.nn.sigmoid(logits)
    softplus = jnp.log1p(jnp.exp(-jnp.abs(logits)))
    ce_pos = jnp.maximum(logits, 0.0) - logits + softplus
    ce_neg = jnp.maximum(logits, 0.0) + softplus
    fl_pos = 0.25 * (1.0 - sp) * (1.0 - sp) * ce_pos
    fl_neg = 0.75 * sp * sp * ce_neg
    return fl_pos, fl_neg


def _one_batch(anch_ref, bbox_ref, conf_ref, gt_ref, bb, n_real, ncol):
    ax1 = anch_ref[0]
    ay1 = anch_ref[1]
    ax2 = anch_ref[2]
    ay2 = anch_ref[3]
    px1 = bbox_ref[bb, 0]
    py1 = bbox_ref[bb, 1]
    px2 = bbox_ref[bb, 2]
    py2 = bbox_ref[bb, 3]
    p = conf_ref[bb]
    gx1 = gt_ref[bb, 0]
    gy1 = gt_ref[bb, 1]
    gx2 = gt_ref[bb, 2]
    gy2 = gt_ref[bb, 3]

    ltx = jnp.maximum(ax1, gx1)
    lty = jnp.maximum(ay1, gy1)
    rbx = jnp.minimum(ax2, gx2)
    rby = jnp.minimum(ay2, gy2)
    wx = jnp.maximum(rbx - ltx, 0.0)
    wy = jnp.maximum(rby - lty, 0.0)
    inter = wx * wy
    area_a = (ax2 - ax1) * (ay2 - ay1)
    area_g = (gx2 - gx1) * (gy2 - gy1)
    iou = inter / (area_a + area_g - inter + 1e-7)

    riota = jax.lax.broadcasted_iota(jnp.int32, (_SL, ncol), 0)
    jiota = jax.lax.broadcasted_iota(jnp.int32, (_SL, ncol), 1)
    gidx = riota * ncol + jiota
    valid = p > 0.0

    best_iou = jnp.max(iou, axis=0)
    giota = jax.lax.broadcasted_iota(jnp.int32, (32, 1, 1), 0)
    bidx = jnp.min(jnp.where(iou == best_iou, giota, 64), axis=0)
    sel = (giota == bidx)
    mx1 = jnp.sum(jnp.where(sel, gx1, 0.0), axis=0)
    my1 = jnp.sum(jnp.where(sel, gy1, 0.0), axis=0)
    mx2 = jnp.sum(jnp.where(sel, gx2, 0.0), axis=0)
    my2 = jnp.sum(jnp.where(sel, gy2, 0.0), axis=0)

    iltx = jnp.maximum(px1, mx1)
    ilty = jnp.maximum(py1, my1)
    irbx = jnp.minimum(px2, mx2)
    irby = jnp.minimum(py2, my2)
    iw = jnp.maximum(irbx - iltx, 0.0)
    ih = jnp.maximum(irby - ilty, 0.0)
    inter2 = iw * ih
    ap = (px2 - px1) * (py2 - py1)
    am = (mx2 - mx1) * (my2 - my1)
    iou2 = inter2 / (ap + am - inter2 + 1e-7)
    pcx = (px1 + px2) * 0.5
    pcy = (py1 + py2) * 0.5
    tcx = (mx1 + mx2) * 0.5
    tcy = (my1 + my2) * 0.5
    rho2 = (pcx - tcx) ** 2 + (pcy - tcy) ** 2
    eltx = jnp.minimum(px1, mx1)
    elty = jnp.minimum(py1, my1)
    erbx = jnp.maximum(px2, mx2)
    erby = jnp.maximum(py2, my2)
    c2 = (erbx - eltx) ** 2 + (erby - elty) ** 2 + 1e-7
    diou = 1.0 - iou2 + rho2 / c2

    fl_pos, fl_neg = _focal_parts(jnp.where(valid, p, 0.5))

    posm = best_iou > 0.5
    posf = posm.astype(_F32)
    loc0 = jnp.sum(diou * posf)
    posfl0 = jnp.sum(fl_pos * posf)
    npos0 = jnp.sum(posf)

    negm = jnp.logical_and(valid, jnp.logical_not(posm))
    flneg = jnp.where(negm, fl_neg, 0.0)
    negp = jnp.where(negm, p, -1.0)

    lmax = _red2(iou, jnp.max)
    gidx_f = gidx.astype(_F32)
    hit = iou == lmax[:, :, None]
    larg = _red2(jnp.where(hit, gidx_f, 3.0e7), jnp.min)
    sel2 = gidx_f == larg[:, :, None]

    def ext(v):
        return _red2(jnp.where(sel2, v, 0.0), jnp.sum)

    m_diou = ext(diou)
    m_p = ext(p)
    m_mask0 = ext(posf)
    m_flpos, m_flneg = _focal_parts(m_p)

    idx_row = _t32(larg)
    ii = jax.lax.broadcasted_iota(jnp.int32, (32, 32), 0)
    jj = jax.lax.broadcasted_iota(jnp.int32, (32, 32), 1)
    dup = jnp.logical_and(idx_row == larg, jj < ii)
    hasdup = jnp.sum(dup.astype(_F32), axis=1, keepdims=True) > 0.0
    fix = jnp.logical_and(jnp.logical_not(hasdup), m_mask0 < 0.5)
    fixf = fix.astype(_F32)

    num_pos = npos0 + jnp.sum(fixf)
    loc = loc0 + jnp.sum(fixf * m_diou)
    posfl = posfl0 + jnp.sum(fixf * m_flpos)
    k = jnp.minimum(3.0 * num_pos, n_real - num_pos)

    p_fix = jnp.where(fix, m_p, -1.0)
    fl_fix = jnp.where(fix, m_flneg, 0.0)
    return dict(loc=loc, posfl=posfl, num_pos=num_pos, k=k,
                negp=negp, flneg=flneg, p_fix=p_fix, fl_fix=fl_fix)


def _neg_topk_sum(st):
    negp, flneg, p_fix, fl_fix, k = (st["negp"], st["flneg"], st["p_fix"],
                                     st["fl_fix"], st["k"])
    v = st["v"]
    m_cnt = (jnp.sum((negp > v).astype(_F32))
             - jnp.sum((p_fix > v).astype(_F32)))
    s_gt = (jnp.sum(flneg * (negp > v).astype(_F32))
            - jnp.sum(fl_fix * (p_fix > v).astype(_F32)))
    c_at = (jnp.sum((negp == v).astype(_F32))
            - jnp.sum((p_fix == v).astype(_F32)))
    s_at = (jnp.sum(flneg * (negp == v).astype(_F32))
            - jnp.sum(fl_fix * (p_fix == v).astype(_F32)))
    rem = k - m_cnt
    corr = jnp.where(rem > 0.5, rem * s_at / jnp.maximum(c_at, 1.0), 0.0)
    return s_gt + corr


def _dl_kernel(anch_ref, bbox_ref, conf_ref, gt_ref,
               out_loss_ref, out_conf_ref, out_loc_ref, acc_ref,
               *, n_real, nsteps, ncol):
    b = pl.program_id(0)

    @pl.when(b == 0)
    def _():
        acc_ref[0] = 0.0
        acc_ref[1] = 0.0
        acc_ref[2] = 0.0

    sts = [_one_batch(anch_ref, bbox_ref, conf_ref, gt_ref, bb, n_real, ncol)
           for bb in range(_BB)]

    def body(_, carry):
        out = []
        for i, st in enumerate(sts):
            lo, hi = carry[2 * i], carry[2 * i + 1]
            mid = 0.5 * (lo + hi)
            cnt = (jnp.sum((st["negp"] >= mid).astype(_F32))
                   - jnp.sum((st["p_fix"] >= mid).astype(_F32)))
            geq = cnt >= st["k"]
            out.append(jnp.where(geq, mid, lo))
            out.append(jnp.where(geq, hi, mid))
        return tuple(out)

    fin = jax.lax.fori_loop(0, _BISECT, body, (0.0, 1.0) * _BB)

    loc = 0.0
    conf = 0.0
    npos = 0.0
    for i, st in enumerate(sts):
        st["v"] = fin[2 * i]
        loc = loc + st["loc"]
        conf = conf + st["posfl"] + _neg_topk_sum(st)
        npos = npos + st["num_pos"]

    acc_ref[0] = acc_ref[0] + loc
    acc_ref[1] = acc_ref[1] + conf
    acc_ref[2] = acc_ref[2] + npos

    @pl.when(b == nsteps - 1)
    def _():
        np_ = jnp.maximum(1.0, acc_ref[2])
        tl = acc_ref[0] / np_
        tc = acc_ref[1] / np_
        one = jnp.ones((1, 1), _F32)
        out_loss_ref[...] = (2.0 * tl + tc) * one
        out_conf_ref[...] = tc * one
        out_loc_ref[...] = tl * one


@jax.jit
def kernel(bbox_pred, conf_pred, anchors, gt_boxes):
    B, N, _ = bbox_pred.shape
    G = gt_boxes.shape[1]
    NP = ((N + 8 * 128 - 1) // (8 * 128)) * (8 * 128)
    NCOL = NP // _SL
    pad = NP - N
    bbox_t = jnp.pad(jnp.transpose(bbox_pred, (0, 2, 1)),
                     ((0, 0), (0, 0), (0, pad))).reshape(B, 4, _SL, NCOL)
    anch_t = jnp.pad(anchors.T, ((0, 0), (0, pad))).reshape(4, _SL, NCOL)
    conf_p = jnp.pad(conf_pred, ((0, 0), (0, pad)),
                     constant_values=-1.0).reshape(B, _SL, NCOL)
    gt_t = jnp.transpose(gt_boxes, (0, 2, 1)).reshape(B, 4, G, 1, 1)

    nsteps = B // _BB
    kern = functools.partial(_dl_kernel, n_real=N, nsteps=nsteps, ncol=NCOL)
    out = pl.pallas_call(
        kern,
        grid=(nsteps,),
        in_specs=[
            pl.BlockSpec((4, _SL, NCOL), lambda b: (0, 0, 0)),
            pl.BlockSpec((_BB, 4, _SL, NCOL), lambda b: (b, 0, 0, 0)),
            pl.BlockSpec((_BB, _SL, NCOL), lambda b: (b, 0, 0)),
            pl.BlockSpec((_BB, 4, G, 1, 1), lambda b: (b, 0, 0, 0, 0)),
        ],
        out_specs=[
            pl.BlockSpec((1, 1), lambda b: (0, 0)),
            pl.BlockSpec((1, 1), lambda b: (0, 0)),
            pl.BlockSpec((1, 1), lambda b: (0, 0)),
        ],
        out_shape=[jax.ShapeDtypeStruct((1, 1), _F32)] * 3,
        scratch_shapes=[
            pltpu.SMEM((4,), _F32),
        ],
        compiler_params=pltpu.CompilerParams(
            dimension_semantics=("arbitrary",)),
    )(anch_t, bbox_t, conf_p, gt_t)
    return (out[0][0, 0], out[1][0, 0], out[2][0, 0])


# speedup vs baseline: 1.2654x; 1.2654x over previous
import functools

import jax
import jax.numpy as jnp
from jax.experimental import pallas as pl
from jax.experimental.pallas import tpu as pltpu

_SL = 8
_BB = 4
_BISECT = 34
_F32 = jnp.float32


def _t32(col):
    i = jax.lax.broadcasted_iota(jnp.int32, (32, 32), 0)
    j = jax.lax.broadcasted_iota(jnp.int32, (32, 32), 1)
    return jnp.sum(jnp.where(i == j, col, 0.0), axis=0, keepdims=True)


def _red2(x, op):
    return op(op(x, axis=2), axis=1, keepdims=True)


def _focal_parts(p):
    logits = jnp.log(p / (1.0 - p + 1e-10))
    sp = jax.nn.sigmoid(logits)
    softplus = jnp.log1p(jnp.exp(-jnp.abs(logits)))
    ce_pos = jnp.maximum(logits, 0.0) - logits + softplus
    ce_neg = jnp.maximum(logits, 0.0) + softplus
    fl_pos = 0.25 * (1.0 - sp) * (1.0 - sp) * ce_pos
    fl_neg = 0.75 * sp * sp * ce_neg
    return fl_pos, fl_neg


def _one_batch(anch_ref, bbox_ref, conf_ref, gt_ref, bb, n_real, ncol):
    ax1 = anch_ref[0]
    ay1 = anch_ref[1]
    ax2 = anch_ref[2]
    ay2 = anch_ref[3]
    px1 = bbox_ref[bb, 0]
    py1 = bbox_ref[bb, 1]
    px2 = bbox_ref[bb, 2]
    py2 = bbox_ref[bb, 3]
    p = conf_ref[bb]
    gx1 = gt_ref[bb, 0]
    gy1 = gt_ref[bb, 1]
    gx2 = gt_ref[bb, 2]
    gy2 = gt_ref[bb, 3]

    ltx = jnp.maximum(ax1, gx1)
    lty = jnp.maximum(ay1, gy1)
    rbx = jnp.minimum(ax2, gx2)
    rby = jnp.minimum(ay2, gy2)
    wx = jnp.maximum(rbx - ltx, 0.0)
    wy = jnp.maximum(rby - lty, 0.0)
    inter = wx * wy
    area_a = (ax2 - ax1) * (ay2 - ay1)
    area_g = (gx2 - gx1) * (gy2 - gy1)
    iou = inter / (area_a + area_g - inter + 1e-7)

    riota = jax.lax.broadcasted_iota(jnp.int32, (_SL, ncol), 0)
    jiota = jax.lax.broadcasted_iota(jnp.int32, (_SL, ncol), 1)
    gidx = riota * ncol + jiota
    valid = p > 0.0

    best_iou = jnp.max(iou, axis=0)
    giota = jax.lax.broadcasted_iota(jnp.int32, (32, 1, 1), 0)
    bidx = jnp.min(jnp.where(iou == best_iou, giota, 64), axis=0)
    sel = (giota == bidx)
    mx1 = jnp.sum(jnp.where(sel, gx1, 0.0), axis=0)
    my1 = jnp.sum(jnp.where(sel, gy1, 0.0), axis=0)
    mx2 = jnp.sum(jnp.where(sel, gx2, 0.0), axis=0)
    my2 = jnp.sum(jnp.where(sel, gy2, 0.0), axis=0)

    iltx = jnp.maximum(px1, mx1)
    ilty = jnp.maximum(py1, my1)
    irbx = jnp.minimum(px2, mx2)
    irby = jnp.minimum(py2, my2)
    iw = jnp.maximum(irbx - iltx, 0.0)
    ih = jnp.maximum(irby - ilty, 0.0)
    inter2 = iw * ih
    ap = (px2 - px1) * (py2 - py1)
    am = (mx2 - mx1) * (my2 - my1)
    iou2 = inter2 / (ap + am - inter2 + 1e-7)
    pcx = (px1 + px2) * 0.5
    pcy = (py1 + py2) * 0.5
    tcx = (mx1 + mx2) * 0.5
    tcy = (my1 + my2) * 0.5
    rho2 = (pcx - tcx) ** 2 + (pcy - tcy) ** 2
    eltx = jnp.minimum(px1, mx1)
    elty = jnp.minimum(py1, my1)
    erbx = jnp.maximum(px2, mx2)
    erby = jnp.maximum(py2, my2)
    c2 = (erbx - eltx) ** 2 + (erby - elty) ** 2 + 1e-7
    diou = 1.0 - iou2 + rho2 / c2

    fl_pos, fl_neg = _focal_parts(jnp.where(valid, p, 0.5))

    posm = best_iou > 0.5
    posf = posm.astype(_F32)
    loc0 = jnp.sum(diou * posf)
    posfl0 = jnp.sum(fl_pos * posf)
    npos0 = jnp.sum(posf)

    negm = jnp.logical_and(valid, jnp.logical_not(posm))
    flneg = jnp.where(negm, fl_neg, 0.0)
    negp = jnp.where(negm, p, -1.0)

    lmax = _red2(iou, jnp.max)
    gidx_f = gidx.astype(_F32)
    hit = iou == lmax[:, :, None]
    larg = _red2(jnp.where(hit, gidx_f, 3.0e7), jnp.min)
    sel2 = gidx_f == larg[:, :, None]

    def ext(v):
        return _red2(jnp.where(sel2, v, 0.0), jnp.sum)

    m_diou = ext(diou)
    m_p = ext(p)
    m_mask0 = ext(posf)
    m_flpos, m_flneg = _focal_parts(m_p)

    idx_row = _t32(larg)
    ii = jax.lax.broadcasted_iota(jnp.int32, (32, 32), 0)
    jj = jax.lax.broadcasted_iota(jnp.int32, (32, 32), 1)
    dup = jnp.logical_and(idx_row == larg, jj < ii)
    hasdup = jnp.sum(dup.astype(_F32), axis=1, keepdims=True) > 0.0
    fix = jnp.logical_and(jnp.logical_not(hasdup), m_mask0 < 0.5)
    fixf = fix.astype(_F32)

    num_pos = npos0 + jnp.sum(fixf)
    loc = loc0 + jnp.sum(fixf * m_diou)
    posfl = posfl0 + jnp.sum(fixf * m_flpos)
    k = jnp.minimum(3.0 * num_pos, n_real - num_pos)

    p_fix = jnp.where(fix, m_p, -1.0)
    fl_fix = jnp.where(fix, m_flneg, 0.0)
    return dict(loc=loc, posfl=posfl, num_pos=num_pos, k=k,
                negp=negp, flneg=flneg, p_fix=p_fix, fl_fix=fl_fix)


def _neg_topk_sum(st):
    negp, flneg, p_fix, fl_fix, k = (st["negp"], st["flneg"], st["p_fix"],
                                     st["fl_fix"], st["k"])
    v = st["v"]
    m_cnt = (jnp.sum((negp > v).astype(_F32))
             - jnp.sum((p_fix > v).astype(_F32)))
    s_gt = (jnp.sum(flneg * (negp > v).astype(_F32))
            - jnp.sum(fl_fix * (p_fix > v).astype(_F32)))
    c_at = (jnp.sum((negp == v).astype(_F32))
            - jnp.sum((p_fix == v).astype(_F32)))
    s_at = (jnp.sum(flneg * (negp == v).astype(_F32))
            - jnp.sum(fl_fix * (p_fix == v).astype(_F32)))
    rem = k - m_cnt
    corr = jnp.where(rem > 0.5, rem * s_at / jnp.maximum(c_at, 1.0), 0.0)
    return s_gt + corr


def _dl_kernel(anch_ref, bbox_ref, conf_ref, gt_ref,
               out_loss_ref, out_conf_ref, out_loc_ref, acc_ref,
               *, n_real, nsteps, ncol):
    b = pl.program_id(0)

    @pl.when(b == 0)
    def _():
        acc_ref[0] = 0.0
        acc_ref[1] = 0.0
        acc_ref[2] = 0.0

    sts = [_one_batch(anch_ref, bbox_ref, conf_ref, gt_ref, bb, n_real, ncol)
           for bb in range(_BB)]

    def body(_, carry):
        out = []
        for i, st in enumerate(sts):
            lo, hi = carry[2 * i], carry[2 * i + 1]
            mid = 0.5 * (lo + hi)
            cnt = (jnp.sum((st["negp"] >= mid).astype(_F32))
                   - jnp.sum((st["p_fix"] >= mid).astype(_F32)))
            geq = cnt >= st["k"]
            out.append(jnp.where(geq, mid, lo))
            out.append(jnp.where(geq, hi, mid))
        return tuple(out)

    fin = jax.lax.fori_loop(0, _BISECT, body, (0.0, 1.0) * _BB)

    loc = 0.0
    conf = 0.0
    npos = 0.0
    for i, st in enumerate(sts):
        st["v"] = fin[2 * i]
        loc = loc + st["loc"]
        conf = conf + st["posfl"] + _neg_topk_sum(st)
        npos = npos + st["num_pos"]

    acc_ref[0] = acc_ref[0] + loc
    acc_ref[1] = acc_ref[1] + conf
    acc_ref[2] = acc_ref[2] + npos

    @pl.when(b == nsteps - 1)
    def _():
        np_ = jnp.maximum(1.0, acc_ref[2])
        tl = acc_ref[0] / np_
        tc = acc_ref[1] / np_
        one = jnp.ones((1, 1), _F32)
        out_loss_ref[...] = (2.0 * tl + tc) * one
        out_conf_ref[...] = tc * one
        out_loc_ref[...] = tl * one


@jax.jit
def kernel(bbox_pred, conf_pred, anchors, gt_boxes):
    B, N, _ = bbox_pred.shape
    G = gt_boxes.shape[1]
    NP = ((N + 8 * 128 - 1) // (8 * 128)) * (8 * 128)
    NCOL = NP // _SL
    pad = NP - N
    bbox_t = jnp.pad(jnp.transpose(bbox_pred, (0, 2, 1)),
                     ((0, 0), (0, 0), (0, pad))).reshape(B, 4, _SL, NCOL)
    anch_t = jnp.pad(anchors.T, ((0, 0), (0, pad))).reshape(4, _SL, NCOL)
    conf_p = jnp.pad(conf_pred, ((0, 0), (0, pad)),
                     constant_values=-1.0).reshape(B, _SL, NCOL)
    gt_t = jnp.transpose(gt_boxes, (0, 2, 1)).reshape(B, 4, G, 1, 1)

    nsteps = B // _BB
    kern = functools.partial(_dl_kernel, n_real=N, nsteps=nsteps, ncol=NCOL)
    out = pl.pallas_call(
        kern,
        grid=(nsteps,),
        in_specs=[
            pl.BlockSpec((4, _SL, NCOL), lambda b: (0, 0, 0)),
            pl.BlockSpec((_BB, 4, _SL, NCOL), lambda b: (b, 0, 0, 0)),
            pl.BlockSpec((_BB, _SL, NCOL), lambda b: (b, 0, 0)),
            pl.BlockSpec((_BB, 4, G, 1, 1), lambda b: (b, 0, 0, 0, 0)),
        ],
        out_specs=[
            pl.BlockSpec((1, 1), lambda b: (0, 0)),
            pl.BlockSpec((1, 1), lambda b: (0, 0)),
            pl.BlockSpec((1, 1), lambda b: (0, 0)),
        ],
        out_shape=[jax.ShapeDtypeStruct((1, 1), _F32)] * 3,
        scratch_shapes=[
            pltpu.SMEM((4,), _F32),
        ],
        compiler_params=pltpu.CompilerParams(
            dimension_semantics=("arbitrary",)),
    )(anch_t, bbox_t, conf_p, gt_t)
    return (out[0][0, 0], out[1][0, 0], out[2][0, 0])
